# baseline (speedup 1.0000x reference)
import jax
import jax.numpy as jnp
from jax.experimental import pallas as pl
from jax.experimental.pallas import tpu as pltpu

NUM_SAMPLES = 100000
NUM_FEATURES = 1024
BATCH = 128
TEMP = 0.05
BLK = 1000

_N_STEPS = NUM_SAMPLES // (2 * BLK)


def _cm_kernel(x_rgb_ref, x_nir_ref, x_tir_ref, tgt_ref,
               fa_rgb_ref, fa_nir_ref, fa_tir_ref,
               fb_rgb_ref, fb_nir_ref, fb_tir_ref,
               o_rgb_ref, o_nir_ref, o_tir_ref,
               xn_rgb, xn_nir, xn_tir, se, gold):
    j = pl.program_id(0)

    @pl.when(j == 0)
    def _init():
        for src, dst in ((x_rgb_ref, xn_rgb), (x_nir_ref, xn_nir),
                         (x_tir_ref, xn_tir)):
            x = src[...]
            n = jnp.sqrt(jnp.sum(x * x, axis=1, keepdims=True))
            dst[...] = x / jnp.maximum(n, 1e-12)
        se[...] = jnp.zeros_like(se)
        gold[...] = jnp.zeros_like(gold)

    inv_t = 1.0 / TEMP
    for half, refs in enumerate(((fa_rgb_ref, fa_nir_ref, fa_tir_ref),
                                 (fb_rgb_ref, fb_nir_ref, fb_tir_ref))):
        col0 = (2 * j + half) * BLK
        cols = jax.lax.broadcasted_iota(jnp.int32, (BATCH, BLK), 1) + col0
        hit = cols == tgt_ref[...]
        for k, (xn, f_ref) in enumerate(((xn_rgb, refs[0]), (xn_nir, refs[1]),
                                         (xn_tir, refs[2]))):
            d = jax.lax.dot_general(
                xn[...].astype(jnp.bfloat16), f_ref[...].astype(jnp.bfloat16),
                (((1,), (1,)), ((), ())),
                preferred_element_type=jnp.float32)
            p = d * inv_t
            se[:, k:k + 1] += jnp.sum(jnp.exp(p - inv_t), axis=1,
                                      keepdims=True)
            gold[:, k:k + 1] += jnp.sum(jnp.where(hit, p, 0.0), axis=1,
                                        keepdims=True)

    @pl.when(j == _N_STEPS - 1)
    def _fini():
        lse = jnp.log(se[...]) + inv_t
        loss = jnp.sum(lse - gold[...], axis=0, keepdims=True) / BATCH
        o_rgb_ref[...] = loss[:, 0:1]
        o_nir_ref[...] = loss[:, 1:2]
        o_tir_ref[...] = loss[:, 2:3]


@jax.jit
def kernel(inputs_rgb, inputs_nir, inputs_tir, targets,
           features_rgb, features_nir, features_tir):
    tgt2d = targets.astype(jnp.int32).reshape(BATCH, 1)

    batch_spec = pl.BlockSpec((BATCH, NUM_FEATURES), lambda j: (0, 0))
    bank_a = pl.BlockSpec((BLK, NUM_FEATURES), lambda j: (2 * j, 0))
    bank_b = pl.BlockSpec((BLK, NUM_FEATURES), lambda j: (2 * j + 1, 0))
    tgt_spec = pl.BlockSpec((BATCH, 1), lambda j: (0, 0))
    out_spec = pl.BlockSpec((1, 1), lambda j: (0, 0))
    scalar = jax.ShapeDtypeStruct((1, 1), jnp.float32)

    o_rgb, o_nir, o_tir = pl.pallas_call(
        _cm_kernel,
        grid=(_N_STEPS,),
        in_specs=[batch_spec, batch_spec, batch_spec, tgt_spec,
                  bank_a, bank_a, bank_a, bank_b, bank_b, bank_b],
        out_specs=[out_spec, out_spec, out_spec],
        out_shape=[scalar, scalar, scalar],
        scratch_shapes=[
            pltpu.VMEM((BATCH, NUM_FEATURES), jnp.float32),
            pltpu.VMEM((BATCH, NUM_FEATURES), jnp.float32),
            pltpu.VMEM((BATCH, NUM_FEATURES), jnp.float32),
            pltpu.VMEM((BATCH, 3), jnp.float32),
            pltpu.VMEM((BATCH, 3), jnp.float32),
        ],
        compiler_params=pltpu.CompilerParams(
            dimension_semantics=("arbitrary",)),
    )(inputs_rgb, inputs_nir, inputs_tir, tgt2d,
      features_rgb, features_nir, features_tir,
      features_rgb, features_nir, features_tir)

    return (o_rgb[0, 0], o_nir[0, 0], o_tir[0, 0])


# speedup vs baseline: 1.0088x; 1.0088x over previous
import functools

import jax
import jax.numpy as jnp
from jax.experimental import pallas as pl
from jax.experimental.pallas import tpu as pltpu

NUM_SAMPLES = 100000
NUM_FEATURES = 1024
BATCH = 128
TEMP = 0.05
BLK = 1000

_N_STEPS = NUM_SAMPLES // BLK


def _cm_kernel(x_rgb_ref, x_nir_ref, x_tir_ref, tgt_ref,
               f_rgb_ref, f_nir_ref, f_tir_ref,
               o_rgb_ref, o_nir_ref, o_tir_ref,
               xn_rgb, xn_nir, xn_tir, se, gold):
    j = pl.program_id(0)

    @pl.when(j == 0)
    def _init():
        for src, dst in ((x_rgb_ref, xn_rgb), (x_nir_ref, xn_nir),
                         (x_tir_ref, xn_tir)):
            x = src[...]
            n = jnp.sqrt(jnp.sum(x * x, axis=1, keepdims=True))
            dst[...] = x / jnp.maximum(n, 1e-12)
        se[...] = jnp.zeros_like(se)
        gold[...] = jnp.zeros_like(gold)

    inv_t = 1.0 / TEMP
    col0 = j * BLK
    cols = jax.lax.broadcasted_iota(jnp.int32, (BATCH, BLK), 1) + col0
    hit = cols == tgt_ref[...]

    for k, (xn, f_ref) in enumerate(((xn_rgb, f_rgb_ref), (xn_nir, f_nir_ref),
                                     (xn_tir, f_tir_ref))):
        d = jax.lax.dot_general(
            xn[...].astype(jnp.bfloat16), f_ref[...].astype(jnp.bfloat16),
            (((1,), (1,)), ((), ())),
            preferred_element_type=jnp.float32)
        p = d * inv_t
        se[:, k:k + 1] += jnp.sum(jnp.exp(p - inv_t), axis=1, keepdims=True)
        gold[:, k:k + 1] += jnp.sum(jnp.where(hit, p, 0.0), axis=1,
                                    keepdims=True)

    @pl.when(j == _N_STEPS - 1)
    def _fini():
        lse = jnp.log(se[...]) + inv_t
        loss = jnp.sum(lse - gold[...], axis=0, keepdims=True) / BATCH
        o_rgb_ref[...] = loss[:, 0:1]
        o_nir_ref[...] = loss[:, 1:2]
        o_tir_ref[...] = loss[:, 2:3]


@jax.jit
def kernel(inputs_rgb, inputs_nir, inputs_tir, targets,
           features_rgb, features_nir, features_tir):
    tgt2d = targets.astype(jnp.int32).reshape(BATCH, 1)

    batch_spec = pl.BlockSpec((BATCH, NUM_FEATURES), lambda j: (0, 0))
    bank_spec = pl.BlockSpec((BLK, NUM_FEATURES), lambda j: (j, 0))
    tgt_spec = pl.BlockSpec((BATCH, 1), lambda j: (0, 0))
    out_spec = pl.BlockSpec((1, 1), lambda j: (0, 0))
    scalar = jax.ShapeDtypeStruct((1, 1), jnp.float32)

    o_rgb, o_nir, o_tir = pl.pallas_call(
        _cm_kernel,
        grid=(_N_STEPS,),
        in_specs=[batch_spec, batch_spec, batch_spec, tgt_spec,
                  bank_spec, bank_spec, bank_spec],
        out_specs=[out_spec, out_spec, out_spec],
        out_shape=[scalar, scalar, scalar],
        scratch_shapes=[
            pltpu.VMEM((BATCH, NUM_FEATURES), jnp.float32),
            pltpu.VMEM((BATCH, NUM_FEATURES), jnp.float32),
            pltpu.VMEM((BATCH, NUM_FEATURES), jnp.float32),
            pltpu.VMEM((BATCH, 3), jnp.float32),
            pltpu.VMEM((BATCH, 3), jnp.float32),
        ],
        compiler_params=pltpu.CompilerParams(
            dimension_semantics=("arbitrary",)),
    )(inputs_rgb, inputs_nir, inputs_tir, tgt2d,
      features_rgb, features_nir, features_tir)

    return (o_rgb[0, 0], o_nir[0, 0], o_tir[0, 0])


# speedup vs baseline: 1.0143x; 1.0055x over previous
import jax
import jax.numpy as jnp
from jax.experimental import pallas as pl
from jax.experimental.pallas import tpu as pltpu

NUM_SAMPLES = 100000
NUM_FEATURES = 1024
BATCH = 128
TEMP = 0.05
BLK = 1000

_N_STEPS = NUM_SAMPLES // BLK


def _cm_kernel(x_rgb_ref, x_nir_ref, x_tir_ref, tgt_ref,
               f_rgb_ref, f_nir_ref, f_tir_ref,
               o_rgb_ref, o_nir_ref, o_tir_ref,
               xn_rgb, xn_nir, xn_tir, se, gold):
    j = pl.program_id(0)

    @pl.when(j == 0)
    def _init():
        for src, dst in ((x_rgb_ref, xn_rgb), (x_nir_ref, xn_nir),
                         (x_tir_ref, xn_tir)):
            x = src[...]
            n = jnp.sqrt(jnp.sum(x * x, axis=1, keepdims=True))
            dst[...] = x / jnp.maximum(n, 1e-12)
        se[...] = jnp.zeros_like(se)
        gold[...] = jnp.zeros_like(gold)

    inv_t = 1.0 / TEMP
    col0 = j * BLK
    cols = jax.lax.broadcasted_iota(jnp.int32, (BATCH, BLK), 1) + col0
    hit = cols == tgt_ref[...]

    for k, (xn, f_ref) in enumerate(((xn_rgb, f_rgb_ref), (xn_nir, f_nir_ref),
                                     (xn_tir, f_tir_ref))):
        d = jax.lax.dot_general(
            xn[...].astype(jnp.bfloat16), f_ref[...].astype(jnp.bfloat16),
            (((1,), (1,)), ((), ())),
            preferred_element_type=jnp.float32)
        p = d * inv_t
        se[:, k:k + 1] += jnp.sum(jnp.exp(p - inv_t), axis=1, keepdims=True)
        gold[:, k:k + 1] += jnp.sum(jnp.where(hit, p, 0.0), axis=1,
                                    keepdims=True)

    @pl.when(j == _N_STEPS - 1)
    def _fini():
        lse = jnp.log(se[...]) + inv_t
        loss = jnp.sum(lse - gold[...], axis=0, keepdims=True) / BATCH
        o_rgb_ref[...] = loss[:, 0:1]
        o_nir_ref[...] = loss[:, 1:2]
        o_tir_ref[...] = loss[:, 2:3]


@jax.jit
def kernel(inputs_rgb, inputs_nir, inputs_tir, targets,
           features_rgb, features_nir, features_tir):
    tgt2d = targets.astype(jnp.int32).reshape(BATCH, 1)

    batch_spec = pl.BlockSpec((BATCH, NUM_FEATURES), lambda j: (0, 0))
    bank_spec = pl.BlockSpec((BLK, NUM_FEATURES), lambda j: (j, 0))
    tgt_spec = pl.BlockSpec((BATCH, 1), lambda j: (0, 0))
    out_spec = pl.BlockSpec((1, 1), lambda j: (0, 0))
    scalar = jax.ShapeDtypeStruct((1, 1), jnp.float32)

    o_rgb, o_nir, o_tir = pl.pallas_call(
        _cm_kernel,
        grid=(_N_STEPS,),
        in_specs=[batch_spec, batch_spec, batch_spec, tgt_spec,
                  bank_spec, bank_spec, bank_spec],
        out_specs=[out_spec, out_spec, out_spec],
        out_shape=[scalar, scalar, scalar],
        scratch_shapes=[
            pltpu.VMEM((BATCH, NUM_FEATURES), jnp.float32),
            pltpu.VMEM((BATCH, NUM_FEATURES), jnp.float32),
            pltpu.VMEM((BATCH, NUM_FEATURES), jnp.float32),
            pltpu.VMEM((BATCH, 3), jnp.float32),
            pltpu.VMEM((BATCH, 3), jnp.float32),
        ],
        compiler_params=pltpu.CompilerParams(
            dimension_semantics=("arbitrary",)),
    )(inputs_rgb, inputs_nir, inputs_tir, tgt2d,
      features_rgb, features_nir, features_tir)

    return (o_rgb[0, 0], o_nir[0, 0], o_tir[0, 0])
